# baseline (speedup 1.0000x reference)
import functools

import jax
import jax.numpy as jnp
from jax import lax
from jax.experimental import pallas as pl
from jax.experimental.pallas import tpu as pltpu
from jax.experimental.pallas import tpu_sc as plsc

_VOCAB = 1000000
_BATCH = 16384
_DIM = 32

_NC = 2
_NS = 16
_NW = _NC * _NS
_BPW = _BATCH // _NW
_HCHUNK = 8


@jax.jit
def kernel(indices, in_vec):
    mesh = plsc.VectorSubcoreMesh(core_axis_name="c", subcore_axis_name="s")

    @functools.partial(
        pl.kernel,
        mesh=mesh,
        out_type=jax.ShapeDtypeStruct((_DIM, _BATCH), jnp.float32),
        scratch_types=[
            pltpu.VMEM((_BPW,), jnp.int32),
            pltpu.VMEM((2, _HCHUNK, _DIM, 128), jnp.float32),
            pltpu.VMEM((_DIM, _BPW), jnp.float32),
            pltpu.SemaphoreType.DMA,
            pltpu.SemaphoreType.DMA,
        ],
        compiler_params=pltpu.CompilerParams(needs_layout_passes=False),
    )
    def gather_kernel(idx_hbm, table_hbm, out_hbm, idx_s, ring_v, block_v,
                      sem_a, sem_b):
        wid = lax.axis_index("s") * _NC + lax.axis_index("c")
        base = wid * _BPW
        pltpu.sync_copy(idx_hbm.at[pl.ds(base, _BPW)], idx_s)

        rows_lo = lax.iota(jnp.int32, 16)
        rows_hi = rows_lo + 16
        sems = (sem_a, sem_b)

        def fire(slot, iv8):
            for j in range(_HCHUNK):
                col0 = pl.multiple_of(
                    lax.shift_right_logical(iv8[j], 7) * 128, 128
                )
                pltpu.make_async_copy(
                    table_hbm.at[:, pl.ds(col0, 128)],
                    ring_v.at[slot, j],
                    sems[slot],
                ).start()

        def drain_extract(slot, h, iv8):
            for j in range(_HCHUNK):
                pltpu.make_async_copy(
                    table_hbm.at[:, pl.ds(0, 128)],
                    ring_v.at[slot, j],
                    sems[slot],
                ).wait()
            for j in range(_HCHUNK):
                col = jnp.full((16,), iv8[j] & 127, dtype=jnp.int32)
                pos = jnp.full((16,), h * _HCHUNK + j, dtype=jnp.int32)
                v_lo = plsc.load_gather(ring_v.at[slot, j], [rows_lo, col])
                v_hi = plsc.load_gather(ring_v.at[slot, j], [rows_hi, col])
                plsc.store_scatter(block_v, [rows_lo, pos], v_lo)
                plsc.store_scatter(block_v, [rows_hi, pos], v_hi)

        n_pairs = _BPW // (2 * _HCHUNK)
        iv0 = idx_s[pl.ds(0, 16)]
        fire(0, iv0[0:8])
        fire(1, iv0[8:16])

        def pair_body(g, iv_cur):
            iv_next = idx_s[pl.ds((g + 1) * 16, 16)]
            drain_extract(0, 2 * g, iv_cur[0:8])
            fire(0, iv_next[0:8])
            drain_extract(1, 2 * g + 1, iv_cur[8:16])
            fire(1, iv_next[8:16])
            return iv_next

        iv_last = lax.fori_loop(0, n_pairs - 1, pair_body, iv0)
        drain_extract(0, 2 * (n_pairs - 1), iv_last[0:8])
        drain_extract(1, 2 * (n_pairs - 1) + 1, iv_last[8:16])

        pltpu.sync_copy(block_v, out_hbm.at[:, pl.ds(base, _BPW)])

    out_t = gather_kernel(indices.astype(jnp.int32), in_vec.T)
    return out_t.T
